# baseline (speedup 1.0000x reference)
import functools

import jax
import jax.numpy as jnp
from jax import lax
from jax.experimental import pallas as pl
from jax.experimental.pallas import tpu as pltpu
from jax.experimental.pallas import tpu_sc as plsc

NUM_CORES = 2
NUM_SUBCORES = 16
LANES = 16
NUM_WORKERS = NUM_CORES * NUM_SUBCORES

BATCH = 16384
EMBED = 32
ROWS_PER_TILE = 32
NTILES = 1000000 // ROWS_PER_TILE
BPW = BATCH // NUM_WORKERS
SLICES = BPW // LANES
CH = 16
NCHUNK = BPW // CH


def _sc_body(uidx_h, midx_h, uemb_h, memb_h, ubias_h, mbias_h,
             partials_h, bsum_h,
             uidx_v, midx_v, ut_v, mt_v,
             utile0_v, utile1_v, mtile0_v, mtile1_v,
             ubias_v, mbias_v, bsum_v, acc_v,
             idx_sh, uidx_s, midx_s,
             sem_u0, sem_u1, sem_m0, sem_m1, sem_ub, sem_mb):
    cid = lax.axis_index("c")
    sid = lax.axis_index("s")
    wid = sid * NUM_CORES + cid
    base = wid * BPW

    pltpu.sync_copy(uidx_h.at[pl.ds(base, BPW)], uidx_v)
    pltpu.sync_copy(midx_h.at[pl.ds(base, BPW)], midx_v)
    pltpu.sync_copy(uidx_v, idx_sh.at[0, sid])
    pltpu.sync_copy(midx_v, idx_sh.at[1, sid])
    pltpu.sync_copy(idx_sh.at[0, sid], uidx_s)
    pltpu.sync_copy(idx_sh.at[1, sid], midx_s)

    cp_ub = pltpu.async_copy(ubias_h.at[uidx_v], ubias_v, sem_ub)
    cp_mb = pltpu.async_copy(mbias_h.at[midx_v], mbias_v, sem_mb)

    def tid_body(j, _):
        sl = pl.ds(j * LANES, LANES)
        ut_v[sl] = lax.shift_right_logical(uidx_v[sl], 5)
        mt_v[sl] = lax.shift_right_logical(midx_v[sl], 5)
        return 0

    lax.fori_loop(0, SLICES, tid_body, 0)

    utiles = (utile0_v, utile1_v)
    mtiles = (mtile0_v, mtile1_v)
    usems = (sem_u0, sem_u1)
    msems = (sem_m0, sem_m1)

    def fire(c, buf):
        sl = pl.ds(c * CH, CH)
        cu = pltpu.async_copy(uemb_h.at[ut_v.at[sl]], utiles[buf], usems[buf])
        cm = pltpu.async_copy(memb_h.at[mt_v.at[sl]], mtiles[buf], msems[buf])
        return cu, cm

    def extract_acc(c, buf, acc):
        ut = utiles[buf]
        mt = mtiles[buf]

        def one(t, acc):
            b = c * CH + t
            ru = lax.rem(uidx_s[b], jnp.int32(ROWS_PER_TILE))
            rm = lax.rem(midx_s[b], jnp.int32(ROWS_PER_TILE))
            su = lax.shift_right_logical(ru, 2)
            sm = lax.shift_right_logical(rm, 2)
            lu = lax.rem(ru, jnp.int32(4)) * EMBED
            lm = lax.rem(rm, jnp.int32(4)) * EMBED
            u0 = ut[t, su, pl.ds(lu, LANES)]
            m0 = mt[t, sm, pl.ds(lm, LANES)]
            u1 = ut[t, su, pl.ds(lu + LANES, LANES)]
            m1 = mt[t, sm, pl.ds(lm + LANES, LANES)]
            return acc + u0 * m0 + u1 * m1

        return lax.fori_loop(0, CH, one, acc)

    acc = jnp.zeros((LANES,), jnp.float32)

    pending = fire(0, 0)
    for c in range(NCHUNK):
        buf = c % 2
        nxt = fire(c + 1, 1 - buf) if c + 1 < NCHUNK else None
        for cp in pending:
            cp.wait()
        acc = extract_acc(c, buf, acc)
        pending = nxt

    acc_v[...] = acc
    pltpu.sync_copy(acc_v, partials_h.at[pl.ds(wid * LANES, LANES)])

    cp_ub.wait()
    cp_mb.wait()

    def bias_body(j, _):
        sl = pl.ds(j * LANES, LANES)
        bsum_v[sl] = ubias_v[sl] + mbias_v[sl]
        return 0

    lax.fori_loop(0, SLICES, bias_body, 0)
    pltpu.sync_copy(bsum_v, bsum_h.at[pl.ds(base, BPW)])


@functools.partial(
    pl.kernel,
    out_type=(
        jax.ShapeDtypeStruct((NUM_WORKERS * LANES,), jnp.float32),
        jax.ShapeDtypeStruct((BATCH,), jnp.float32),
    ),
    mesh=plsc.VectorSubcoreMesh(core_axis_name="c", subcore_axis_name="s"),
    scratch_types=(
        pltpu.VMEM((BPW,), jnp.int32),
        pltpu.VMEM((BPW,), jnp.int32),
        pltpu.VMEM((BPW,), jnp.int32),
        pltpu.VMEM((BPW,), jnp.int32),
        pltpu.VMEM((CH, 8, 128), jnp.float32),
        pltpu.VMEM((CH, 8, 128), jnp.float32),
        pltpu.VMEM((CH, 8, 128), jnp.float32),
        pltpu.VMEM((CH, 8, 128), jnp.float32),
        pltpu.VMEM((BPW,), jnp.float32),
        pltpu.VMEM((BPW,), jnp.float32),
        pltpu.VMEM((BPW,), jnp.float32),
        pltpu.VMEM((LANES,), jnp.float32),
        pltpu.VMEM_SHARED((2, NUM_SUBCORES, BPW), jnp.int32),
        pltpu.SMEM((BPW,), jnp.int32),
        pltpu.SMEM((BPW,), jnp.int32),
        pltpu.SemaphoreType.DMA,
        pltpu.SemaphoreType.DMA,
        pltpu.SemaphoreType.DMA,
        pltpu.SemaphoreType.DMA,
        pltpu.SemaphoreType.DMA,
        pltpu.SemaphoreType.DMA,
    ),
)
def _sc_gather(uidx_h, midx_h, uemb_h, memb_h, ubias_h, mbias_h,
               partials_h, bsum_h, *scratch):
    _sc_body(uidx_h, midx_h, uemb_h, memb_h, ubias_h, mbias_h,
             partials_h, bsum_h, *scratch)


def _tc_combine_body(part_ref, bsum_ref, out_ref):
    total = jnp.sum(part_ref[...])
    out_ref[...] = jax.nn.sigmoid(bsum_ref[...] + total)


_tc_combine = pl.pallas_call(
    _tc_combine_body,
    out_shape=jax.ShapeDtypeStruct((BATCH // 128, 128), jnp.float32),
)


def kernel(inputs, user_embedding, user_bias, movie_embedding, movie_bias):
    u_idx = inputs[:, 0]
    m_idx = inputs[:, 1]
    partials, bsum = _sc_gather(
        u_idx, m_idx,
        user_embedding.reshape(NTILES, 8, 128),
        movie_embedding.reshape(NTILES, 8, 128),
        user_bias.reshape(-1), movie_bias.reshape(-1))
    out = _tc_combine(partials.reshape(4, 128), bsum.reshape(BATCH // 128, 128))
    return out.reshape(BATCH, 1)
